# baseline (speedup 1.0000x reference)
import functools

import jax
import jax.numpy as jnp
from jax import lax
from jax.experimental import pallas as pl
from jax.experimental.pallas import tpu as pltpu
from jax.experimental.pallas import tpu_sc as plsc

CONF_THR = 0.5
DIST_THR = 2.0
NUM_K = 512

_NC = 2
_NS = 16
_LANES = 16
_ACC_ROWS = 520
_TRASH = NUM_K


def _conf_idx_body(ev_ref, lab_ref, conf_ref, idx_ref, cnt_ref):
    i = pl.program_id(0)
    ev = ev_ref[...]
    conf = jnp.sum(ev, axis=1) * (1.0 / ev.shape[1])
    conf_ref[...] = conf
    high = conf > CONF_THR
    labs = lab_ref[...]
    idx_ref[...] = jnp.where(high, labs, jnp.int32(_TRASH))
    b = labs.shape[0]
    kio = lax.broadcasted_iota(jnp.int32, (b, NUM_K), 1)
    w = jnp.where((labs[:, None] == kio) & high[:, None], 1.0, 0.0)

    @pl.when(i == 0)
    def _():
        cnt_ref[...] = jnp.zeros_like(cnt_ref)

    cnt_ref[...] += jnp.sum(w, axis=0)[:, None]


def _make_sc_segsum(n, d):
    dh = d // _NC
    rows_s = n // _NS
    chunk = 256
    nchunks = rows_s // chunk
    mesh = plsc.VectorSubcoreMesh(core_axis_name="c", subcore_axis_name="s")

    @functools.partial(
        pl.kernel,
        mesh=mesh,
        out_type=jax.ShapeDtypeStruct((_NC * _NS, NUM_K, dh), jnp.float32),
        scratch_types=[
            pltpu.VMEM((chunk, dh), jnp.float32),
            pltpu.VMEM((chunk,), jnp.int32),
            pltpu.VMEM((_ACC_ROWS, dh), jnp.float32),
        ],
    )
    def segsum(emb_hbm, idx_hbm, sums_hbm, emb_v, idx_v, acc_v):
        cid = lax.axis_index("c")
        sid = lax.axis_index("s")
        col0 = cid * dh
        zero16 = jnp.zeros((_LANES,), jnp.float32)
        io16 = lax.iota(jnp.int32, _LANES)
        cols = [io16 + j * _LANES for j in range(dh // _LANES)]

        def zero_acc(r, _):
            for j in range(dh // _LANES):
                acc_v[r, pl.ds(j * _LANES, _LANES)] = zero16
            return 0
        lax.fori_loop(0, _ACC_ROWS, zero_acc, 0)

        for t in range(nchunks):
            r0 = sid * rows_s + t * chunk
            pltpu.sync_copy(emb_hbm.at[pl.ds(r0, chunk), pl.ds(col0, dh)],
                            emb_v)
            pltpu.sync_copy(idx_hbm.at[pl.ds(r0, chunk)], idx_v)

            def row_body(g, _):
                base = g * _LANES
                labs16 = idx_v[pl.ds(base, _LANES)]
                for u in range(_LANES):
                    lab = labs16[u]
                    for j in range(dh // _LANES):
                        v = emb_v[base + u, pl.ds(j * _LANES, _LANES)]
                        plsc.addupdate(acc_v.at[lab, pl.ds(j * _LANES, _LANES)], v)
                return 0
            lax.fori_loop(0, chunk // _LANES, row_body, 0)

        widx = cid * _NS + sid
        pltpu.sync_copy(acc_v.at[pl.ds(0, NUM_K)], sums_hbm.at[widx])

    return segsum


def _phase_b_body(emb_ref, conf_ref, lab_ref, psums_ref, cnt_ref, rand_ref,
                  nl_ref, md_ref, ctr_ref, c2_ref):
    i = pl.program_id(0)

    @pl.when(i == 0)
    def _():
        dh = psums_ref.shape[2]
        counts = cnt_ref[...]
        cdiv = jnp.maximum(counts, 1.0)
        nonempty = counts > 0.0
        c2 = jnp.zeros((1, NUM_K), jnp.float32)
        for h in range(_NC):
            sums_h = psums_ref[h * _NS]
            for s in range(1, _NS):
                sums_h = sums_h + psums_ref[h * _NS + s]
            ctr_h = jnp.where(nonempty, sums_h / cdiv,
                              rand_ref[:, h * dh:(h + 1) * dh])
            ctr_ref[:, h * dh:(h + 1) * dh] = ctr_h
            c2 = c2 + jnp.sum(ctr_h * ctr_h, axis=1)[None, :]
        c2_ref[...] = c2

    centers = ctr_ref[...]
    c2 = c2_ref[...]
    emb = emb_ref[...]
    x2 = jnp.sum(emb * emb, axis=1)[:, None]
    dot = lax.dot_general(
        emb, centers, (((1,), (1,)), ((), ())),
        preferred_element_type=jnp.float32)
    d2 = jnp.maximum(x2 + c2 - 2.0 * dot, 0.0)
    mind2 = jnp.min(d2, axis=1)
    min_d = jnp.sqrt(mind2)
    kio = lax.broadcasted_iota(jnp.int32, d2.shape, 1)
    near = jnp.min(jnp.where(d2 == mind2[:, None], kio, NUM_K), axis=1)
    near = near.astype(jnp.int32)
    reassigned = jnp.where(min_d > DIST_THR, jnp.int32(-1), near)
    hard = jnp.logical_not(conf_ref[...] > CONF_THR)
    nl_ref[...] = jnp.where(hard, reassigned, lab_ref[...])
    md_ref[...] = min_d


def kernel(embeddings, evidence_strengths, current_labels, num_clusters):
    n, d = embeddings.shape
    l = evidence_strengths.shape[1]
    ev2 = evidence_strengths.reshape(n, l)
    ba = 512
    bb = 512
    rand_centers = jax.random.normal(jax.random.key(42), (NUM_K, d),
                                     jnp.float32)

    conf, idx, counts = pl.pallas_call(
        _conf_idx_body,
        grid=(n // ba,),
        in_specs=[
            pl.BlockSpec((ba, l), lambda i: (i, 0)),
            pl.BlockSpec((ba,), lambda i: (i,)),
        ],
        out_specs=[
            pl.BlockSpec((ba,), lambda i: (i,)),
            pl.BlockSpec((ba,), lambda i: (i,)),
            pl.BlockSpec((NUM_K, 1), lambda i: (0, 0)),
        ],
        out_shape=[
            jax.ShapeDtypeStruct((n,), jnp.float32),
            jax.ShapeDtypeStruct((n,), jnp.int32),
            jax.ShapeDtypeStruct((NUM_K, 1), jnp.float32),
        ],
        compiler_params=pltpu.CompilerParams(
            dimension_semantics=("arbitrary",)),
    )(ev2, current_labels)

    psums = _make_sc_segsum(n, d)(embeddings, idx)

    new_labels, min_d = pl.pallas_call(
        _phase_b_body,
        grid=(n // bb,),
        in_specs=[
            pl.BlockSpec((bb, d), lambda i: (i, 0)),
            pl.BlockSpec((bb,), lambda i: (i,)),
            pl.BlockSpec((bb,), lambda i: (i,)),
            pl.BlockSpec((_NC * _NS, NUM_K, d // _NC), lambda i: (0, 0, 0)),
            pl.BlockSpec((NUM_K, 1), lambda i: (0, 0)),
            pl.BlockSpec((NUM_K, d), lambda i: (0, 0)),
        ],
        out_specs=[
            pl.BlockSpec((bb,), lambda i: (i,)),
            pl.BlockSpec((bb,), lambda i: (i,)),
        ],
        out_shape=[
            jax.ShapeDtypeStruct((n,), jnp.int32),
            jax.ShapeDtypeStruct((n,), jnp.float32),
        ],
        scratch_shapes=[
            pltpu.VMEM((NUM_K, d), jnp.float32),
            pltpu.VMEM((1, NUM_K), jnp.float32),
        ],
        compiler_params=pltpu.CompilerParams(
            dimension_semantics=("arbitrary",)),
    )(embeddings, conf, current_labels, psums, counts, rand_centers)

    return new_labels, min_d, conf


# speedup vs baseline: 1.0916x; 1.0916x over previous
import functools

import jax
import jax.numpy as jnp
from jax import lax
from jax.experimental import pallas as pl
from jax.experimental.pallas import tpu as pltpu
from jax.experimental.pallas import tpu_sc as plsc

CONF_THR = 0.5
DIST_THR = 2.0
NUM_K = 512

_NC = 2
_NS = 16
_LANES = 16
_ACC_ROWS = 520
_TRASH = NUM_K


def _conf_idx_body(ev_ref, lab_ref, conf_ref, idx_ref, cnt_ref):
    i = pl.program_id(0)
    ev = ev_ref[...]
    conf = jnp.sum(ev, axis=1) * (1.0 / ev.shape[1])
    conf_ref[...] = conf
    high = conf > CONF_THR
    labs = lab_ref[...]
    idx_ref[...] = jnp.where(high, labs, jnp.int32(_TRASH))
    b = labs.shape[0]
    kio = lax.broadcasted_iota(jnp.int32, (b, NUM_K), 1)
    w = jnp.where((labs[:, None] == kio) & high[:, None], 1.0, 0.0)

    @pl.when(i == 0)
    def _():
        cnt_ref[...] = jnp.zeros_like(cnt_ref)

    cnt_ref[...] += jnp.sum(w, axis=0)[:, None]


def _make_sc_segsum(n, d):
    dh = d // _NC
    rows_s = n // _NS
    chunk = 128
    nchunks = rows_s // chunk
    mesh = plsc.VectorSubcoreMesh(core_axis_name="c", subcore_axis_name="s")

    @functools.partial(
        pl.kernel,
        mesh=mesh,
        out_type=jax.ShapeDtypeStruct((_NC * _NS, NUM_K, dh), jnp.float32),
        scratch_types=[
            pltpu.VMEM((chunk, dh), jnp.float32),
            pltpu.VMEM((chunk,), jnp.int32),
            pltpu.VMEM((_ACC_ROWS, dh), jnp.float32),
        ],
    )
    def segsum(emb_hbm, idx_hbm, sums_hbm, emb_v, idx_v, acc_v):
        cid = lax.axis_index("c")
        sid = lax.axis_index("s")
        col0 = cid * dh
        zero16 = jnp.zeros((_LANES,), jnp.float32)
        nj = dh // _LANES

        def zero_acc(r, _):
            for j in range(nj):
                acc_v[r, pl.ds(j * _LANES, _LANES)] = zero16
            return 0
        lax.fori_loop(0, _ACC_ROWS, zero_acc, 0)

        for t in range(nchunks):
            r0 = sid * rows_s + t * chunk
            pltpu.sync_copy(emb_hbm.at[pl.ds(r0, chunk), pl.ds(col0, dh)],
                            emb_v)
            pltpu.sync_copy(idx_hbm.at[pl.ds(r0, chunk)], idx_v)

            def row_body(g, _):
                base = g * _LANES
                labs16 = idx_v[pl.ds(base, _LANES)]
                labs = [labs16[u] for u in range(_LANES)]
                for u in range(_LANES):
                    lab = labs[u]
                    vs = [emb_v[base + u, pl.ds(j * _LANES, _LANES)]
                          for j in range(nj)]
                    for j in range(nj):
                        plsc.addupdate(acc_v.at[lab, pl.ds(j * _LANES, _LANES)],
                                       vs[j])
                return 0
            lax.fori_loop(0, chunk // _LANES, row_body, 0)

        widx = cid * _NS + sid
        pltpu.sync_copy(acc_v.at[pl.ds(0, NUM_K)], sums_hbm.at[widx])

    return segsum


def _phase_b_body(emb_ref, conf_ref, lab_ref, psums_ref, cnt_ref, rand_ref,
                  nl_ref, md_ref, ctr_ref, c2_ref):
    i = pl.program_id(0)

    @pl.when(i == 0)
    def _():
        dh = psums_ref.shape[2]
        counts = cnt_ref[...]
        cdiv = jnp.maximum(counts, 1.0)
        nonempty = counts > 0.0
        c2 = jnp.zeros((1, NUM_K), jnp.float32)
        for h in range(_NC):
            sums_h = psums_ref[h * _NS]
            for s in range(1, _NS):
                sums_h = sums_h + psums_ref[h * _NS + s]
            ctr_h = jnp.where(nonempty, sums_h / cdiv,
                              rand_ref[:, h * dh:(h + 1) * dh])
            ctr_ref[:, h * dh:(h + 1) * dh] = ctr_h
            c2 = c2 + jnp.sum(ctr_h * ctr_h, axis=1)[None, :]
        c2_ref[...] = c2

    centers = ctr_ref[...]
    c2 = c2_ref[...]
    emb = emb_ref[...]
    x2 = jnp.sum(emb * emb, axis=1)[:, None]
    dot = lax.dot_general(
        emb, centers, (((1,), (1,)), ((), ())),
        preferred_element_type=jnp.float32)
    d2 = jnp.maximum(x2 + c2 - 2.0 * dot, 0.0)
    mind2 = jnp.min(d2, axis=1)
    min_d = jnp.sqrt(mind2)
    kio = lax.broadcasted_iota(jnp.int32, d2.shape, 1)
    near = jnp.min(jnp.where(d2 == mind2[:, None], kio, NUM_K), axis=1)
    near = near.astype(jnp.int32)
    reassigned = jnp.where(min_d > DIST_THR, jnp.int32(-1), near)
    hard = jnp.logical_not(conf_ref[...] > CONF_THR)
    nl_ref[...] = jnp.where(hard, reassigned, lab_ref[...])
    md_ref[...] = min_d


def kernel(embeddings, evidence_strengths, current_labels, num_clusters):
    n, d = embeddings.shape
    l = evidence_strengths.shape[1]
    ev2 = evidence_strengths.reshape(n, l)
    ba = 512
    bb = 512
    rand_centers = jax.random.normal(jax.random.key(42), (NUM_K, d),
                                     jnp.float32)

    conf, idx, counts = pl.pallas_call(
        _conf_idx_body,
        grid=(n // ba,),
        in_specs=[
            pl.BlockSpec((ba, l), lambda i: (i, 0)),
            pl.BlockSpec((ba,), lambda i: (i,)),
        ],
        out_specs=[
            pl.BlockSpec((ba,), lambda i: (i,)),
            pl.BlockSpec((ba,), lambda i: (i,)),
            pl.BlockSpec((NUM_K, 1), lambda i: (0, 0)),
        ],
        out_shape=[
            jax.ShapeDtypeStruct((n,), jnp.float32),
            jax.ShapeDtypeStruct((n,), jnp.int32),
            jax.ShapeDtypeStruct((NUM_K, 1), jnp.float32),
        ],
        compiler_params=pltpu.CompilerParams(
            dimension_semantics=("arbitrary",)),
    )(ev2, current_labels)

    psums = _make_sc_segsum(n, d)(embeddings, idx)

    new_labels, min_d = pl.pallas_call(
        _phase_b_body,
        grid=(n // bb,),
        in_specs=[
            pl.BlockSpec((bb, d), lambda i: (i, 0)),
            pl.BlockSpec((bb,), lambda i: (i,)),
            pl.BlockSpec((bb,), lambda i: (i,)),
            pl.BlockSpec((_NC * _NS, NUM_K, d // _NC), lambda i: (0, 0, 0)),
            pl.BlockSpec((NUM_K, 1), lambda i: (0, 0)),
            pl.BlockSpec((NUM_K, d), lambda i: (0, 0)),
        ],
        out_specs=[
            pl.BlockSpec((bb,), lambda i: (i,)),
            pl.BlockSpec((bb,), lambda i: (i,)),
        ],
        out_shape=[
            jax.ShapeDtypeStruct((n,), jnp.int32),
            jax.ShapeDtypeStruct((n,), jnp.float32),
        ],
        scratch_shapes=[
            pltpu.VMEM((NUM_K, d), jnp.float32),
            pltpu.VMEM((1, NUM_K), jnp.float32),
        ],
        compiler_params=pltpu.CompilerParams(
            dimension_semantics=("arbitrary",)),
    )(embeddings, conf, current_labels, psums, counts, rand_centers)

    return new_labels, min_d, conf
